# baseline (speedup 1.0000x reference)
import functools

import jax
import jax.numpy as jnp
from jax import lax
from jax.experimental import pallas as pl
from jax.experimental.pallas import tpu as pltpu
from jax.experimental.pallas import tpu_sc as plsc

BATCH = 1024
MAXLEN = 200
EMBED_DIM = 64
VOCAB = 1000000
LANES = 16
NUM_WORKERS = 32
SEQ_PER_WORKER = BATCH // NUM_WORKERS
IDS_PER_WORKER = SEQ_PER_WORKER * MAXLEN
ROWS_PAD = 208
NBLK = ROWS_PAD // LANES
CHUNKS = ((0, 104), (104, 96))
NBUF = 2


def _embed_body(x_hbm, tok_hbm, post_hbm, out_hbm,
                pos_v, idx_all, gidx_all, rows0, rows1, outv,
                g0, g1, s0):
    nc = 2
    wid = lax.axis_index("s") * nc + lax.axis_index("c")
    base = wid * SEQ_PER_WORKER

    pltpu.sync_copy(post_hbm, pos_v)
    pltpu.sync_copy(x_hbm.at[pl.ds(wid * IDS_PER_WORKER, IDS_PER_WORKER)],
                    idx_all.at[pl.ds(0, IDS_PER_WORKER)])

    def halve(c, carry):
        sl = pl.ds(c * LANES, LANES)
        gidx_all[sl] = lax.shift_right_logical(
            lax.clamp(jnp.int32(0), idx_all[sl], jnp.int32(VOCAB - 1)), 1)
        return carry
    lax.fori_loop(0, (IDS_PER_WORKER + LANES) // LANES, halve, 0)

    bufs = ((rows0, g0), (rows1, g1))
    iota = lax.iota(jnp.int32, LANES)

    def start_seq(i, buf):
        rows, gsem = buf
        cps = []
        for off, ln in CHUNKS:
            cps.append(pltpu.async_copy(
                tok_hbm.at[gidx_all.at[pl.ds(i * MAXLEN + off, ln)]],
                rows.at[pl.ds(off, ln)], gsem))
        return cps

    inflight = [None] * NBUF
    store_cp = None
    inflight[0] = start_seq(0, bufs[0])
    for i in range(SEQ_PER_WORKER):
        cur = i % NBUF
        nxt = (i + 1) % NBUF
        if i + 1 < SEQ_PER_WORKER:
            inflight[nxt] = start_seq(i + 1, bufs[nxt])
        for cp in inflight[cur]:
            cp.wait()
        if store_cp is not None:
            store_cp.wait()
        rows, _ = bufs[cur]

        def sel_block(rb, carry):
            row16 = rb * LANES + iota
            par = lax.bitwise_and(
                idx_all[pl.ds(i * MAXLEN + rb * LANES, LANES)], 1)
            colbase = par * EMBED_DIM

            def col(c, carry2):
                vals = plsc.load_gather(rows, [row16, colbase + c])
                pv = pos_v[pl.ds(c * MAXLEN + rb * LANES, LANES)]
                plsc.store_scatter(outv, [row16, iota * 0 + c], vals + pv)
                return carry2

            lax.fori_loop(0, EMBED_DIM, col, 0)
            return carry

        lax.fori_loop(0, NBLK, sel_block, 0)
        store_cp = pltpu.async_copy(
            outv.at[pl.ds(0, MAXLEN)],
            out_hbm.at[pl.ds((base + i) * MAXLEN, MAXLEN)], s0)
    store_cp.wait()


@jax.jit
def _embed(x, token_table, pos_table):
    tok_pairs = token_table.reshape(VOCAB // 2, 2 * EMBED_DIM)
    pos_t = pos_table.T.reshape(MAXLEN * EMBED_DIM)
    x_flat = x.reshape(BATCH * MAXLEN)
    mesh = plsc.VectorSubcoreMesh(core_axis_name="c", subcore_axis_name="s")
    run = functools.partial(
        pl.kernel, mesh=mesh,
        out_type=jax.ShapeDtypeStruct((BATCH * MAXLEN, EMBED_DIM),
                                      jnp.float32),
        scratch_types=[
            pltpu.VMEM((MAXLEN * EMBED_DIM,), jnp.float32),
            pltpu.VMEM((IDS_PER_WORKER + LANES,), jnp.int32),
            pltpu.VMEM((IDS_PER_WORKER + LANES,), jnp.int32),
            pltpu.VMEM((ROWS_PAD, 2 * EMBED_DIM), jnp.float32),
            pltpu.VMEM((ROWS_PAD, 2 * EMBED_DIM), jnp.float32),
            pltpu.VMEM((ROWS_PAD, EMBED_DIM), jnp.float32),
            pltpu.SemaphoreType.DMA,
            pltpu.SemaphoreType.DMA,
            pltpu.SemaphoreType.DMA,
        ],
        compiler_params=pltpu.CompilerParams(use_tc_tiling_on_sc=True,
                                             needs_layout_passes=False),
    )(_embed_body)
    out = run(x_flat, tok_pairs, pos_t)
    return out.reshape(BATCH, MAXLEN, EMBED_DIM)


def kernel(x, token_table, pos_table):
    return _embed(x.astype(jnp.int32), token_table, pos_table)


# speedup vs baseline: 1.5472x; 1.5472x over previous
import functools

import jax
import jax.numpy as jnp
from jax import lax
from jax.experimental import pallas as pl
from jax.experimental.pallas import tpu as pltpu
from jax.experimental.pallas import tpu_sc as plsc

BATCH = 1024
MAXLEN = 200
EMBED_DIM = 64
VOCAB = 1000000
LANES = 16
NUM_WORKERS = 32
SEQ_PER_WORKER = BATCH // NUM_WORKERS
IDS_PER_WORKER = SEQ_PER_WORKER * MAXLEN
CHUNKS = ((0, 104), (104, 96))
NBUF = 2


def _embed_body(x_hbm, tok_hbm, post_hbm, out_hbm,
                pos_v, idx_all, rows0, rows1, g0, g1, s0, s1):
    nc = 2
    wid = lax.axis_index("s") * nc + lax.axis_index("c")
    base = wid * SEQ_PER_WORKER

    pltpu.sync_copy(post_hbm, pos_v)
    pltpu.sync_copy(x_hbm.at[pl.ds(wid * IDS_PER_WORKER, IDS_PER_WORKER)],
                    idx_all)

    bufs = ((rows0, g0, s0), (rows1, g1, s1))

    def start_seq(i, buf):
        rows, gsem, _ = buf
        cps = []
        for off, ln in CHUNKS:
            cps.append(pltpu.async_copy(
                tok_hbm.at[idx_all.at[pl.ds(i * MAXLEN + off, ln)]],
                rows.at[pl.ds(off, ln)], gsem))
        return cps

    inflight = [None] * NBUF
    store_cp = [None] * NBUF
    inflight[0] = start_seq(0, bufs[0])
    for i in range(SEQ_PER_WORKER):
        cur = i % NBUF
        nxt = (i + 1) % NBUF
        if i + 1 < SEQ_PER_WORKER:
            if store_cp[nxt] is not None:
                store_cp[nxt].wait()
            inflight[nxt] = start_seq(i + 1, bufs[nxt])
        for cp in inflight[cur]:
            cp.wait()
        rows = bufs[cur][0]

        def add_pos(r, carry):
            for j in range(EMBED_DIM // LANES):
                sl = pl.ds(j * LANES, LANES)
                plsc.addupdate(rows.at[r, sl], pos_v[r, sl])
            return carry

        lax.fori_loop(0, MAXLEN, add_pos, 0)
        store_cp[cur] = pltpu.async_copy(
            rows, out_hbm.at[pl.ds((base + i) * MAXLEN, MAXLEN)],
            bufs[cur][2])
    for cp in store_cp:
        if cp is not None:
            cp.wait()


@jax.jit
def _embed(x, token_table, pos_table):
    x_flat = x.reshape(BATCH * MAXLEN)
    mesh = plsc.VectorSubcoreMesh(core_axis_name="c", subcore_axis_name="s")
    run = functools.partial(
        pl.kernel, mesh=mesh,
        out_type=jax.ShapeDtypeStruct((BATCH * MAXLEN, EMBED_DIM),
                                      jnp.float32),
        scratch_types=[
            pltpu.VMEM((MAXLEN, EMBED_DIM), jnp.float32),
            pltpu.VMEM((IDS_PER_WORKER,), jnp.int32),
            pltpu.VMEM((MAXLEN, EMBED_DIM), jnp.float32),
            pltpu.VMEM((MAXLEN, EMBED_DIM), jnp.float32),
            pltpu.SemaphoreType.DMA,
            pltpu.SemaphoreType.DMA,
            pltpu.SemaphoreType.DMA,
            pltpu.SemaphoreType.DMA,
        ],
        compiler_params=pltpu.CompilerParams(use_tc_tiling_on_sc=False),
    )(_embed_body)
    out = run(x_flat, token_table, pos_table)
    return out.reshape(BATCH, MAXLEN, EMBED_DIM)


def kernel(x, token_table, pos_table):
    return _embed(x.astype(jnp.int32), token_table, pos_table)
